# baseline (speedup 1.0000x reference)
import functools

import jax
import jax.numpy as jnp
import numpy as np
from jax import lax
from jax.experimental import pallas as pl
from jax.experimental.pallas import tpu as pltpu
from jax.experimental.pallas import tpu_sc as plsc

_ALPHA = 4


def kernel(frames):
    B, T, C, H, W = frames.shape
    S = T // _ALPHA
    ROW = C * H * W

    idx = np.linspace(0.0, T - 1, S).astype(np.int64)
    assert np.array_equal(idx, (np.arange(S) * (T - 1)) // (S - 1))

    NW = 32
    n_rows = B * T
    n_slow = B * S
    fast_per_w = n_rows // NW
    slow_per_w = n_slow // NW

    NCHUNK = 4
    CHUNK = ROW // NCHUNK
    NBUF = 3
    n_fast_iter = fast_per_w * NCHUNK
    n_iter = n_fast_iter + slow_per_w * NCHUNK

    mesh = plsc.VectorSubcoreMesh(core_axis_name="c", subcore_axis_name="s")

    @functools.partial(
        pl.kernel,
        out_type=(
            jax.ShapeDtypeStruct((n_slow * ROW,), jnp.float32),
            jax.ShapeDtypeStruct((n_rows * ROW,), jnp.float32),
        ),
        mesh=mesh,
        scratch_types=[
            pltpu.VMEM((CHUNK,), jnp.float32),
            pltpu.VMEM((CHUNK,), jnp.float32),
            pltpu.VMEM((CHUNK,), jnp.float32),
            pltpu.SemaphoreType.DMA,
            pltpu.SemaphoreType.DMA,
            pltpu.SemaphoreType.DMA,
            pltpu.SemaphoreType.DMA,
            pltpu.SemaphoreType.DMA,
            pltpu.SemaphoreType.DMA,
        ],
    )
    def pathway(src_hbm, slow_hbm, fast_hbm, b0, b1, b2, l0, l1, l2, s0, s1, s2):
        buf = (b0, b1, b2)
        lsem = (l0, l1, l2)
        ssem = (s0, s1, s2)
        wid = lax.axis_index("s") * 2 + lax.axis_index("c")

        def unit(i):
            if i < n_fast_iter:
                k, c = divmod(i, NCHUNK)
                off = (wid * fast_per_w + k) * ROW + c * CHUNK
                return off, fast_hbm, off
            k, c = divmod(i - n_fast_iter, NCHUNK)
            r = wid * slow_per_w + k
            b = r // S
            j = r % S
            src_row = b * T + (j * (T - 1)) // (S - 1)
            return src_row * ROW + c * CHUNK, slow_hbm, r * ROW + c * CHUNK

        loads = [None] * NBUF
        stores = [None] * NBUF

        def start_load(i):
            soff, _, _ = unit(i)
            loads[i % NBUF] = pltpu.async_copy(
                src_hbm.at[pl.ds(soff, CHUNK)], buf[i % NBUF], lsem[i % NBUF]
            )

        def start_store(i):
            _, dref, doff = unit(i)
            stores[i % NBUF] = pltpu.async_copy(
                buf[i % NBUF], dref.at[pl.ds(doff, CHUNK)], ssem[i % NBUF]
            )

        for i in range(NBUF - 1):
            start_load(i)
        for i in range(n_iter):
            nxt = i + NBUF - 1
            if nxt < n_iter:
                if stores[nxt % NBUF] is not None:
                    stores[nxt % NBUF].wait()
                start_load(nxt)
            loads[i % NBUF].wait()
            start_store(i)
        for i in range(n_iter - NBUF, n_iter):
            stores[i % NBUF].wait()

    slow, fast = pathway(frames.reshape(-1))
    return slow.reshape(B, S, C, H, W), fast.reshape(B, T, C, H, W)


# speedup vs baseline: 3.3864x; 3.3864x over previous
import functools

import jax
import jax.numpy as jnp
import numpy as np
from jax import lax
from jax.experimental import pallas as pl
from jax.experimental.pallas import tpu as pltpu
from jax.experimental.pallas import tpu_sc as plsc

_ALPHA = 4


def kernel(frames):
    B, T, C, H, W = frames.shape
    S = T // _ALPHA

    idx = np.linspace(0.0, T - 1, S).astype(np.int64)
    assert np.array_equal(idx, (np.arange(S) * (T - 1)) // (S - 1))

    NW = 32
    n_fast_planes = B * T * C
    n_slow_planes = B * S * C
    fast_per_w = n_fast_planes // NW
    slow_per_w = n_slow_planes // NW
    n_iter = fast_per_w + slow_per_w
    NBUF = 2

    mesh = plsc.VectorSubcoreMesh(core_axis_name="c", subcore_axis_name="s")

    @functools.partial(
        pl.kernel,
        out_type=(
            jax.ShapeDtypeStruct((B, S, C, H, W), jnp.float32),
            jax.ShapeDtypeStruct((B, T, C, H, W), jnp.float32),
        ),
        mesh=mesh,
        scratch_types=[
            pltpu.VMEM((H, W), jnp.float32),
            pltpu.VMEM((H, W), jnp.float32),
            pltpu.SemaphoreType.DMA,
            pltpu.SemaphoreType.DMA,
            pltpu.SemaphoreType.DMA,
            pltpu.SemaphoreType.DMA,
        ],
    )
    def pathway(src_hbm, slow_hbm, fast_hbm, b0, b1, l0, l1, s0, s1):
        buf = (b0, b1)
        lsem = (l0, l1)
        ssem = (s0, s1)
        wid = lax.axis_index("s") * 2 + lax.axis_index("c")

        def unit(i):
            if i < fast_per_w:
                p = wid * fast_per_w + i
                f, c = divmod(p, C)
                b, t = divmod(f, T)
                return src_hbm.at[b, t, c], fast_hbm.at[b, t, c]
            q = wid * slow_per_w + (i - fast_per_w)
            r, c = divmod(q, C)
            b, j = divmod(r, S)
            t = (j * (T - 1)) // (S - 1)
            return src_hbm.at[b, t, c], slow_hbm.at[b, j, c]

        loads = [None] * NBUF
        stores = [None] * NBUF

        def start_load(i):
            src, _ = unit(i)
            loads[i % NBUF] = pltpu.async_copy(src, buf[i % NBUF], lsem[i % NBUF])

        def start_store(i):
            _, dst = unit(i)
            stores[i % NBUF] = pltpu.async_copy(buf[i % NBUF], dst, ssem[i % NBUF])

        start_load(0)
        for i in range(n_iter):
            nxt = i + 1
            if nxt < n_iter:
                if stores[nxt % NBUF] is not None:
                    stores[nxt % NBUF].wait()
                start_load(nxt)
            loads[i % NBUF].wait()
            start_store(i)
        for i in range(n_iter - NBUF, n_iter):
            stores[i % NBUF].wait()

    return pathway(frames)


# speedup vs baseline: 3.7285x; 1.1010x over previous
import functools

import jax
import jax.numpy as jnp
import numpy as np
from jax import lax
from jax.experimental import pallas as pl
from jax.experimental.pallas import tpu as pltpu
from jax.experimental.pallas import tpu_sc as plsc

_ALPHA = 4


def kernel(frames):
    B, T, C, H, W = frames.shape
    S = T // _ALPHA

    idx = np.linspace(0.0, T - 1, S).astype(np.int64)
    assert np.array_equal(idx, (np.arange(S) * (T - 1)) // (S - 1))

    NW = 32
    WPB = NW // B
    FPW = T // WPB
    SPW = S // WPB
    n_slow_iter = SPW * C
    n_iter = FPW * C
    NBUF = 2

    mesh = plsc.VectorSubcoreMesh(core_axis_name="c", subcore_axis_name="s")

    @functools.partial(
        pl.kernel,
        out_type=(
            jax.ShapeDtypeStruct((B, S, C, H, W), jnp.float32),
            jax.ShapeDtypeStruct((B, T, C, H, W), jnp.float32),
        ),
        mesh=mesh,
        scratch_types=[
            pltpu.VMEM((H, W), jnp.float32),
            pltpu.VMEM((H, W), jnp.float32),
            pltpu.SemaphoreType.DMA,
            pltpu.SemaphoreType.DMA,
            pltpu.SemaphoreType.DMA,
            pltpu.SemaphoreType.DMA,
            pltpu.SemaphoreType.DMA,
            pltpu.SemaphoreType.DMA,
        ],
    )
    def pathway(src_hbm, slow_hbm, fast_hbm, b0, b1, l0, l1, f0, f1, w0, w1):
        buf = (b0, b1)
        lsem = (l0, l1)
        fsem = (f0, f1)
        wsem = (w0, w1)
        wid = lax.axis_index("s") * 2 + lax.axis_index("c")
        b = wid // WPB
        q = wid % WPB
        a0 = (2 * q * (T - 1)) // (S - 1) - FPW * q
        a1 = ((2 * q + 1) * (T - 1)) // (S - 1) - FPW * q

        def unit(i):
            if i < n_slow_iter:
                sj, c = divmod(i, C)
                j = SPW * q + sj
                t = (j * (T - 1)) // (S - 1)
                return src_hbm.at[b, t, c], fast_hbm.at[b, t, c], slow_hbm.at[b, j, c]
            fpos, c = divmod(i - n_slow_iter, C)
            u = fpos + (a0 <= fpos).astype(jnp.int32)
            v = u + (a1 <= u).astype(jnp.int32)
            t = FPW * q + v
            return src_hbm.at[b, t, c], fast_hbm.at[b, t, c], None

        loads = [None] * NBUF
        fstores = [None] * NBUF
        sstores = [None] * NBUF

        def start_load(i):
            src, _, _ = unit(i)
            loads[i % NBUF] = pltpu.async_copy(src, buf[i % NBUF], lsem[i % NBUF])

        def start_stores(i):
            _, fdst, sdst = unit(i)
            fstores[i % NBUF] = pltpu.async_copy(buf[i % NBUF], fdst, fsem[i % NBUF])
            if sdst is not None:
                sstores[i % NBUF] = pltpu.async_copy(buf[i % NBUF], sdst, wsem[i % NBUF])

        def wait_stores(i):
            fstores[i % NBUF].wait()
            if i < n_slow_iter:
                sstores[i % NBUF].wait()

        start_load(0)
        for i in range(n_iter):
            nxt = i + 1
            if nxt < n_iter:
                if nxt - NBUF >= 0:
                    wait_stores(nxt - NBUF)
                start_load(nxt)
            loads[i % NBUF].wait()
            start_stores(i)
        for i in range(n_iter - NBUF, n_iter):
            wait_stores(i)

    return pathway(frames)
